# baseline (speedup 1.0000x reference)
import functools

import jax
import jax.numpy as jnp
from jax import lax
from jax.experimental import pallas as pl
from jax.experimental.pallas import tpu as pltpu
from jax.experimental.pallas import tpu_sc as plsc

NUM_HEADS = 16
SEQ = 2048
MAX_DISTANCE = 128
NBIAS = 2 * MAX_DISTANCE + 1
ROWS_PER_WORKER = SEQ // 2
BLOCKS_PER_WORKER = ROWS_PER_WORKER // 8
NTILES = 50
WT_PAD = 384
WCOL = 560

RUNT = 16


def _column_specs(core_val):
    r_lo, r_hi = core_val * 128, core_val * 128 + 128
    cols = []
    for c in range(16):
        lo_start = max(32 + 16 * c, r_lo)
        for r0 in range(lo_start, r_hi, 16):
            cols.append((r0, c, "lo"))
        hi_end = min(16 * c - 16, r_hi)
        for r0 in range(r_lo, hi_end, 16):
            cols.append((r0, c, "hi"))
    return cols


_PERM_DNUMS = lax.GatherDimensionNumbers(
    offset_dims=(), collapsed_slice_dims=(0,), start_index_map=(0,))


def _lane_perm(vec, idx):
    return lax.gather(vec, idx[:, None], _PERM_DNUMS, (1,),
                      mode=lax.GatherScatterMode.PROMISE_IN_BOUNDS)


@functools.partial(
    pl.kernel,
    out_type=jax.ShapeDtypeStruct((NUM_HEADS, SEQ, SEQ), jnp.float32),
    mesh=plsc.VectorSubcoreMesh(core_axis_name="c", subcore_axis_name="s"),
    scratch_types=[
        pltpu.VMEM((WCOL,), jnp.float32),
        pltpu.VMEM((8 * NTILES, 128), jnp.float32),
        pltpu.VMEM((8 * RUNT + 8, 128), jnp.float32),
        pltpu.VMEM((8 * RUNT + 8, 128), jnp.float32),
        pltpu.SemaphoreType.DMA,
    ],
)
def _bias_kernel(wt_hbm, out_hbm, wcol, tiles, lorun, hirun, sem):
    core = lax.axis_index("c")
    sub = lax.axis_index("s")
    h = sub
    block_base = core * BLOCKS_PER_WORKER

    pltpu.sync_copy(wt_hbm.at[pl.ds(pl.multiple_of(h * WT_PAD, 8), WT_PAD)],
                    wcol.at[pl.ds(144, WT_PAD)])

    lanes = lax.iota(jnp.int32, 16)
    zeros = lanes * 0
    lo_vec = _lane_perm(wcol[pl.ds(144, 16)], zeros)
    hi_vec = _lane_perm(wcol[pl.ds(400, 16)], zeros)
    for p in range(0, 144, 16):
        wcol[pl.ds(p, 16)] = lo_vec
    for p in range(400, WCOL, 16):
        wcol[pl.ds(p, 16)] = hi_vec

    def build_tile(t, carry):
        for j in range(8):
            base = t * 8 + (16 - j)
            sigma = lax.bitwise_and(base, 15)
            a0 = base - sigma
            rot = lax.bitwise_and(lanes + sigma, 15)
            first = lanes < (16 - sigma)
            chunks = [
                _lane_perm(wcol[pl.ds(pl.multiple_of(a0 + 16 * u, 16), 16)], rot)
                for u in range(9)
            ]
            for u in range(8):
                vals = jnp.where(first, chunks[u], chunks[u + 1])
                tiles[t * 8 + j, pl.ds(16 * u, 16)] = vals
        return carry

    def fill_lorun(i, carry):
        a = _lane_perm(wcol[pl.ds(0, 16)], zeros)
        b = _lane_perm(wcol[pl.ds(16, 16)], zeros)
        lo_v = jnp.where(lanes < 16, a, b)
        for j in range(8):
            for u in range(8):
                lorun[i * 8 + j, pl.ds(16 * u, 16)] = lo_v
        return carry

    def fill_hirun(i, carry):
        a = _lane_perm(wcol[pl.ds(544, 16)], zeros)
        b = _lane_perm(wcol[pl.ds(528, 16)], zeros)
        hi_v = jnp.where(lanes < 16, a, b)
        for j in range(8):
            for u in range(8):
                hirun[i * 8 + j, pl.ds(16 * u, 16)] = hi_v
        return carry

    lax.fori_loop(0, RUNT, fill_lorun, 0)
    lax.fori_loop(0, RUNT, fill_hirun, 0)
    lax.fori_loop(0, NTILES, build_tile, 0)

    def column_copy(spec, start):
        r0, c, kind = spec
        src = (lorun if kind == "lo" else hirun).at[pl.ds(0, 8 * RUNT), :]
        dst = out_hbm.at[h, pl.ds(r0 * 8, 8 * RUNT), pl.ds(128 * c, 128)]
        if start:
            return pltpu.async_copy(src, dst, sem)
        return pltpu.make_async_copy(src, dst, sem)

    for core_val in (0, 1):
        @pl.when(core == core_val)
        def _(core_val=core_val):
            cols = _column_specs(core_val)
            wave1, wave2, rest = cols[:16], cols[16:32], cols[32:]
            for sp in wave1:
                column_copy(sp, True)
            for sp in wave2:
                column_copy(sp, True)
            for sp in wave1:
                column_copy(sp, False).wait()
            for b in range(0, len(rest), 8):
                batch = rest[b:b + 8]
                for sp in batch:
                    column_copy(sp, True)
                for sp in batch:
                    column_copy(sp, False).wait()
            for sp in wave2:
                column_copy(sp, False).wait()

    def issue_block(r):
        cb = jnp.clip(lax.shift_right_arithmetic(r - 16, 4), 0, 12)
        copies = []
        for k in range(4):
            c = cb + k
            tidx = jnp.clip(32 - r + 16 * c, 0, NTILES - 1)
            src = tiles.at[pl.ds(pl.multiple_of(tidx * 8, 8), 8), :]
            dst = out_hbm.at[h,
                             pl.ds(pl.multiple_of(r * 8, 8), 8),
                             pl.ds(pl.multiple_of(128 * c, 128), 128)]
            copies.append(pltpu.async_copy(src, dst, sem))
        return copies

    issue_block(block_base)

    def blocks(rb, carry):
        copies = issue_block(block_base + rb + 1)
        for cp in copies:
            cp.wait()
        return carry

    lax.fori_loop(0, BLOCKS_PER_WORKER - 1, blocks, 0)
    for k in range(4):
        pltpu.make_async_copy(
            tiles.at[pl.ds(0, 8), :],
            out_hbm.at[h, pl.ds(pl.multiple_of(block_base * 8, 8), 8),
                       pl.ds(128 * k, 128)],
            sem,
        ).wait()


def kernel(x, relative_bias_weight):
    del x
    wt = jnp.pad(relative_bias_weight.T, ((0, 0), (0, WT_PAD - NBIAS)))
    return _bias_kernel(wt.reshape(-1))


# speedup vs baseline: 1.1209x; 1.1209x over previous
import functools

import jax
import jax.numpy as jnp
from jax import lax
from jax.experimental import pallas as pl
from jax.experimental.pallas import tpu as pltpu
from jax.experimental.pallas import tpu_sc as plsc

NUM_HEADS = 16
SEQ = 2048
MAX_DISTANCE = 128
NBIAS = 2 * MAX_DISTANCE + 1
ROWS_PER_WORKER = SEQ // 2
BLOCKS_PER_WORKER = ROWS_PER_WORKER // 8
NTILES = 50
WT_PAD = 384
WCOL = 560

_PERM_DNUMS = lax.GatherDimensionNumbers(
    offset_dims=(), collapsed_slice_dims=(0,), start_index_map=(0,))


def _lane_perm(vec, idx):
    return lax.gather(vec, idx[:, None], _PERM_DNUMS, (1,),
                      mode=lax.GatherScatterMode.PROMISE_IN_BOUNDS)


@functools.partial(
    pl.kernel,
    out_type=jax.ShapeDtypeStruct((NUM_HEADS, SEQ, SEQ), jnp.float32),
    mesh=plsc.VectorSubcoreMesh(core_axis_name="c", subcore_axis_name="s"),
    scratch_types=[
        pltpu.VMEM((WCOL,), jnp.float32),
        pltpu.VMEM((8 * NTILES, 128), jnp.float32),
        pltpu.SemaphoreType.DMA,
    ],
)
def _bias_kernel(wt_hbm, out_hbm, wcol, tiles, sem):
    core = lax.axis_index("c")
    sub = lax.axis_index("s")
    h = sub
    block_base = core * BLOCKS_PER_WORKER

    pltpu.sync_copy(wt_hbm.at[pl.ds(pl.multiple_of(h * WT_PAD, 8), WT_PAD)],
                    wcol.at[pl.ds(144, WT_PAD)])

    lanes = lax.iota(jnp.int32, 16)
    zeros = lanes * 0
    lo_vec = _lane_perm(wcol[pl.ds(144, 16)], zeros)
    hi_vec = _lane_perm(wcol[pl.ds(400, 16)], zeros)
    for p in range(0, 144, 16):
        wcol[pl.ds(p, 16)] = lo_vec
    for p in range(400, WCOL, 16):
        wcol[pl.ds(p, 16)] = hi_vec

    def build_tile(t, carry):
        for j in range(8):
            base = t * 8 + (16 - j)
            sigma = lax.bitwise_and(base, 15)
            a0 = base - sigma
            rot = lax.bitwise_and(lanes + sigma, 15)
            first = lanes < (16 - sigma)
            chunks = [
                _lane_perm(wcol[pl.ds(pl.multiple_of(a0 + 16 * u, 16), 16)], rot)
                for u in range(9)
            ]
            for u in range(8):
                vals = jnp.where(first, chunks[u], chunks[u + 1])
                tiles[t * 8 + j, pl.ds(16 * u, 16)] = vals
        return carry

    build_tile(0, 0)
    build_tile(NTILES - 1, 0)

    def tile_copy(r, c, tidx):
        src = tiles.at[pl.ds(pl.multiple_of(tidx * 8, 8), 8), :]
        dst = out_hbm.at[h,
                         pl.ds(pl.multiple_of(r * 8, 8), 8),
                         pl.ds(pl.multiple_of(128 * c, 128), 128)]
        return pltpu.async_copy(src, dst, sem)

    def drain_one(k):
        pltpu.make_async_copy(
            tiles.at[pl.ds(0, 8), :],
            out_hbm.at[h, pl.ds(pl.multiple_of(block_base * 8, 8), 8),
                       pl.ds(128 * k, 128)],
            sem,
        ).wait()

    def block_cb(r):
        return jnp.clip(lax.shift_right_arithmetic(r - 16, 4), 0, 12)

    def issue_const_block(r):
        cb = block_cb(r)
        copies = []
        for k in range(12):
            c = k + jnp.where(cb <= k, 4, 0)
            tidx = jnp.clip(32 - r + 16 * c, 0, NTILES - 1)
            copies.append(tile_copy(r, c, tidx))
        return copies

    issue_const_block(block_base)

    def const_blocks(rb, carry):
        copies = issue_const_block(block_base + rb + 1)

        @pl.when(rb < NTILES - 2)
        def _():
            build_tile(rb + 1, 0)

        for cp in copies:
            cp.wait()
        return carry

    lax.fori_loop(0, BLOCKS_PER_WORKER - 1, const_blocks, 0)
    for k in range(12):
        drain_one(k)

    def issue_band_block(r):
        cb = block_cb(r)
        copies = []
        for k in range(4):
            c = cb + k
            tidx = jnp.clip(32 - r + 16 * c, 0, NTILES - 1)
            copies.append(tile_copy(r, c, tidx))
        return copies

    issue_band_block(block_base)

    def band_blocks(rb, carry):
        copies = issue_band_block(block_base + rb + 1)
        for cp in copies:
            cp.wait()
        return carry

    lax.fori_loop(0, BLOCKS_PER_WORKER - 1, band_blocks, 0)
    for k in range(4):
        drain_one(k)


def kernel(x, relative_bias_weight):
    del x
    wt = jnp.pad(relative_bias_weight.T, ((0, 0), (0, WT_PAD - NBIAS)))
    return _bias_kernel(wt.reshape(-1))


# speedup vs baseline: 1.1369x; 1.0143x over previous
import functools

import jax
import jax.numpy as jnp
from jax import lax
from jax.experimental import pallas as pl
from jax.experimental.pallas import tpu as pltpu
from jax.experimental.pallas import tpu_sc as plsc

NUM_HEADS = 16
SEQ = 2048
MAX_DISTANCE = 128
NBIAS = 2 * MAX_DISTANCE + 1
ROWS_PER_WORKER = SEQ // 2
BLOCKS_PER_WORKER = ROWS_PER_WORKER // 8
NTILES = 50
WT_PAD = 384
WCOL = 560

_PERM_DNUMS = lax.GatherDimensionNumbers(
    offset_dims=(), collapsed_slice_dims=(0,), start_index_map=(0,))


def _lane_perm(vec, idx):
    return lax.gather(vec, idx[:, None], _PERM_DNUMS, (1,),
                      mode=lax.GatherScatterMode.PROMISE_IN_BOUNDS)


@functools.partial(
    pl.kernel,
    out_type=jax.ShapeDtypeStruct((NUM_HEADS, SEQ, SEQ), jnp.float32),
    mesh=plsc.VectorSubcoreMesh(core_axis_name="c", subcore_axis_name="s"),
    scratch_types=[
        pltpu.VMEM((WCOL,), jnp.float32),
        pltpu.VMEM((8 * NTILES, 128), jnp.float32),
        pltpu.VMEM((8, 1024), jnp.float32),
        pltpu.SemaphoreType.DMA,
    ],
)
def _bias_kernel(wt_hbm, out_hbm, wcol, tiles, constw, sem):
    core = lax.axis_index("c")
    sub = lax.axis_index("s")
    h = sub
    block_base = core * BLOCKS_PER_WORKER

    pltpu.sync_copy(wt_hbm.at[pl.ds(pl.multiple_of(h * WT_PAD, 8), WT_PAD)],
                    wcol.at[pl.ds(144, WT_PAD)])

    lanes = lax.iota(jnp.int32, 16)
    zeros = lanes * 0
    lo_vec = _lane_perm(wcol[pl.ds(144, 16)], zeros)
    hi_vec = _lane_perm(wcol[pl.ds(400, 16)], zeros)
    for p in range(0, 144, 16):
        wcol[pl.ds(p, 16)] = lo_vec
    for p in range(400, WCOL, 16):
        wcol[pl.ds(p, 16)] = hi_vec

    def build_tile(t, carry):
        for j in range(8):
            base = t * 8 + (16 - j)
            sigma = lax.bitwise_and(base, 15)
            a0 = base - sigma
            rot = lax.bitwise_and(lanes + sigma, 15)
            first = lanes < (16 - sigma)
            chunks = [
                _lane_perm(wcol[pl.ds(pl.multiple_of(a0 + 16 * u, 16), 16)], rot)
                for u in range(9)
            ]
            for u in range(8):
                vals = jnp.where(first, chunks[u], chunks[u + 1])
                tiles[t * 8 + j, pl.ds(16 * u, 16)] = vals
        return carry

    def fill_constw(i, carry):
        lo_v = jnp.where(lanes < 16, _lane_perm(wcol[pl.ds(0, 16)], zeros),
                         _lane_perm(wcol[pl.ds(16, 16)], zeros))
        hi_v = jnp.where(lanes < 16, _lane_perm(wcol[pl.ds(544, 16)], zeros),
                         _lane_perm(wcol[pl.ds(528, 16)], zeros))
        for j in range(8):
            constw[j, pl.ds(pl.multiple_of(i * 16, 16), 16)] = lo_v
            constw[j, pl.ds(pl.multiple_of(512 + i * 16, 16), 16)] = hi_v
        return carry

    lax.fori_loop(0, 32, fill_constw, 0)

    build_tile(0, 0)
    build_tile(NTILES - 1, 0)

    def tile_copy(r, c, tidx):
        src = tiles.at[pl.ds(pl.multiple_of(tidx * 8, 8), 8), :]
        dst = out_hbm.at[h,
                         pl.ds(pl.multiple_of(r * 8, 8), 8),
                         pl.ds(pl.multiple_of(128 * c, 128), 128)]
        return pltpu.async_copy(src, dst, sem)

    def drain_one(k):
        pltpu.make_async_copy(
            tiles.at[pl.ds(0, 8), :],
            out_hbm.at[h, pl.ds(pl.multiple_of(block_base * 8, 8), 8),
                       pl.ds(128 * k, 128)],
            sem,
        ).wait()

    def block_cb(r):
        return jnp.clip(lax.shift_right_arithmetic(r - 16, 4), 0, 12)

    def issue_const_block(r):
        cb = block_cb(r)
        qb = jnp.clip(lax.shift_right_arithmetic(cb, 2), 0, 2)
        copies = []
        for k in range(2):
            is_hi = qb <= k
            q = k + jnp.where(is_hi, 2, 0)
            src = constw.at[:, pl.ds(
                pl.multiple_of(jnp.where(is_hi, 512, 0), 128), 512)]
            dst = out_hbm.at[h,
                             pl.ds(pl.multiple_of(r * 8, 8), 8),
                             pl.ds(pl.multiple_of(512 * q, 128), 512)]
            copies.append(pltpu.async_copy(src, dst, sem))
        return copies

    issue_const_block(block_base)

    def const_blocks(rb, carry):
        copies = issue_const_block(block_base + rb + 1)

        @pl.when(rb < NTILES - 2)
        def _():
            build_tile(rb + 1, 0)

        for cp in copies:
            cp.wait()
        return carry

    lax.fori_loop(0, BLOCKS_PER_WORKER - 1, const_blocks, 0)
    for k in range(2):
        pltpu.make_async_copy(
            constw.at[:, pl.ds(0, 512)],
            out_hbm.at[h, pl.ds(pl.multiple_of(block_base * 8, 8), 8),
                       pl.ds(512 * k, 512)],
            sem,
        ).wait()

    def issue_band_block(r):
        qb = jnp.clip(lax.shift_right_arithmetic(block_cb(r), 2), 0, 2)
        copies = []
        for k in range(8):
            c = qb * 4 + k
            tidx = jnp.clip(32 - r + 16 * c, 0, NTILES - 1)
            copies.append(tile_copy(r, c, tidx))
        return copies

    issue_band_block(block_base)

    def band_blocks(rb, carry):
        copies = issue_band_block(block_base + rb + 1)
        for cp in copies:
            cp.wait()
        return carry

    lax.fori_loop(0, BLOCKS_PER_WORKER - 1, band_blocks, 0)
    for k in range(8):
        drain_one(k)


def kernel(x, relative_bias_weight):
    del x
    wt = jnp.pad(relative_bias_weight.T, ((0, 0), (0, WT_PAD - NBIAS)))
    return _bias_kernel(wt.reshape(-1))
